# baseline (speedup 1.0000x reference)
import functools

import jax
import jax.numpy as jnp
from jax import lax
from jax.experimental import pallas as pl
from jax.experimental.pallas import tpu as pltpu
from jax.experimental.pallas import tpu_sc as plsc

N = 100000
E = 1600000
H = 64
B = 64

NCHUNK = 8
CH = N // NCHUNK
TPR = 784
CHP = 16 * TPR
G = 512
SB = 2000
ET32 = E // 32
STAG = 2560
BCAP = ((ET32 + G - 1) // G) * G

_mesh = plsc.VectorSubcoreMesh(core_axis_name="c", subcore_axis_name="s")
_sc_params = pltpu.CompilerParams(needs_layout_passes=False,
                                  use_tc_tiling_on_sc=False)


def _bin_body(dst_hbm, src_hbm,
              bsrc, bdst, counts, deg_out0, deg_out1,
              dstbuf, srcbuf,
              st_s0, st_s1, st_s2, st_s3, st_s4, st_s5, st_s6, st_s7,
              st_d0, st_d1, st_d2, st_d3, st_d4, st_d5, st_d6, st_d7,
              ones_v, zbuf, cntv, deg_sp, sem):
    c = lax.axis_index("c")
    s = lax.axis_index("s")
    wid = s * 2 + c
    iota = lax.iota(jnp.int32, 16)
    st_s = [st_s0, st_s1, st_s2, st_s3, st_s4, st_s5, st_s6, st_s7]
    st_d = [st_d0, st_d1, st_d2, st_d3, st_d4, st_d5, st_d6, st_d7]

    def _fill(j, _):
        zbuf[pl.ds(j * 16, 16)] = jnp.zeros((16,), jnp.float32)
        ones_v[pl.ds(j * 16, 16)] = jnp.ones((16,), jnp.float32)
        return 0

    lax.fori_loop(0, SB // 16, _fill, 0)

    NBLK = N // SB
    for k in range((NBLK + 15) // 16):
        blk = 16 * k

        @pl.when(blk + s < NBLK)
        def _z():
            pltpu.sync_copy(zbuf, deg_sp.at[pl.ds((blk + s) * SB, SB)])

    plsc.subcore_barrier()

    ebase = wid * ET32

    def vloop(v, carry):
        d = dstbuf[pl.ds(v * 16, 16)]
        sv = srcbuf[pl.ds(v * 16, 16)]
        out = []
        for k in range(NCHUNK):
            lo_v = jnp.full((16,), k * CH, jnp.int32)
            hi_v = jnp.full((16,), (k + 1) * CH, jnp.int32)
            m = (d >= lo_v) & (d < hi_v)
            mi = jnp.where(m, 1, 0)
            cs = plsc.cumsum(mi)
            pos = jnp.full((16,), carry[k], jnp.int32) + cs - 1
            plsc.store_scatter(st_d[k], [pos], d - lo_v, mask=m)
            plsc.store_scatter(st_s[k], [pos], sv, mask=m)
            out.append(carry[k] + cs[15])
        return tuple(out)

    def block(b, carry):
        pltpu.sync_copy(dst_hbm.at[pl.ds(ebase + b * SB, SB)], dstbuf)
        pltpu.sync_copy(src_hbm.at[pl.ds(ebase + b * SB, SB)], srcbuf)
        pltpu.sync_copy(ones_v, deg_sp.at[dstbuf], add=True)
        leads = lax.fori_loop(0, SB // 16, vloop, carry[:NCHUNK])
        new = []
        ngs = []
        for k in range(NCHUNK):
            cnt = leads[k]
            ng = cnt // G

            def flush(g, _, k=k):
                off = (carry[NCHUNK + k] + g) * G
                pltpu.sync_copy(st_s[k].at[pl.ds(g * G, G)],
                                bsrc.at[wid, k, pl.ds(off, G)])
                pltpu.sync_copy(st_d[k].at[pl.ds(g * G, G)],
                                bdst.at[wid, k, pl.ds(off, G)])
                return 0

            lax.fori_loop(0, ng, flush, 0)

            def mv(j, _, k=k):
                st_s[k][pl.ds(j * 16, 16)] = st_s[k][pl.ds(ng * G + j * 16, 16)]
                st_d[k][pl.ds(j * 16, 16)] = st_d[k][pl.ds(ng * G + j * 16, 16)]
                return 0

            lax.fori_loop(0, G // 16, mv, 0)
            new.append(cnt - ng * G)
            ngs.append(carry[NCHUNK + k] + ng)
        return tuple(new) + tuple(ngs)

    carry = lax.fori_loop(0, ET32 // SB, block,
                          (jnp.int32(0),) * (2 * NCHUNK))

    pad_src = iota + jnp.full((16,), wid * 64, jnp.int32)
    totals = []
    for k in range(NCHUNK):
        lead = carry[k]
        ngroups = carry[NCHUNK + k]
        pad_dst = (iota & 7) + jnp.full((16,), CH + 8 * (wid & 1), jnp.int32)

        @pl.when(lead > 0)
        def _final(k=k, lead=lead, ngroups=ngroups, pad_dst=pad_dst):
            for j in range(G // 16):
                st_d[k][pl.ds(lead + j * 16, 16)] = pad_dst
                st_s[k][pl.ds(lead + j * 16, 16)] = pad_src
            off = ngroups * G
            pltpu.sync_copy(st_s[k].at[pl.ds(0, G)], bsrc.at[wid, k, pl.ds(off, G)])
            pltpu.sync_copy(st_d[k].at[pl.ds(0, G)], bdst.at[wid, k, pl.ds(off, G)])

        totals.append((ngroups + jnp.where(lead > 0, 1, 0)) * G)

    cv = jnp.full((16,), 0, jnp.int32)
    for k in range(NCHUNK):
        cv = jnp.where(iota == k, jnp.full((16,), totals[k], jnp.int32), cv)
    cntv[...] = cv
    pltpu.sync_copy(cntv, counts.at[wid])

    plsc.subcore_barrier()
    for k in range((NBLK + 15) // 16):
        blk = 16 * k

        @pl.when(blk + s < NBLK)
        def _stage():
            pltpu.sync_copy(deg_sp.at[pl.ds((blk + s) * SB, SB)], zbuf)

        @pl.when((blk + s < NBLK) & (c == 0))
        def _wb0():
            pltpu.sync_copy(zbuf, deg_out0.at[pl.ds((blk + s) * SB, SB)])

        @pl.when((blk + s < NBLK) & (c == 1))
        def _wb1():
            pltpu.sync_copy(zbuf, deg_out1.at[pl.ds((blk + s) * SB, SB)])


_bin_kernel = functools.partial(
    pl.kernel,
    out_type=[
        jax.ShapeDtypeStruct((32, NCHUNK, BCAP), jnp.int32),
        jax.ShapeDtypeStruct((32, NCHUNK, BCAP), jnp.int32),
        jax.ShapeDtypeStruct((32, 16), jnp.int32),
        jax.ShapeDtypeStruct((N,), jnp.float32),
        jax.ShapeDtypeStruct((N,), jnp.float32),
    ],
    mesh=_mesh,
    scratch_types=[
        pltpu.VMEM((SB,), jnp.int32),
        pltpu.VMEM((SB,), jnp.int32),
    ] + [pltpu.VMEM((STAG,), jnp.int32)] * 16 + [
        pltpu.VMEM((SB,), jnp.float32),
        pltpu.VMEM((SB,), jnp.float32),
        pltpu.VMEM((16,), jnp.int32),
        pltpu.VMEM_SHARED((N,), jnp.float32),
        pltpu.SemaphoreType.DMA,
    ],
    compiler_params=_sc_params,
)(_bin_body)


def _layer_body(hs, bsrc, bdst, counts, out,
                gsrc2, gdst2, rows2, cv, acc, gsem0, gsem1, ssem0, ssem1):
    c = lax.axis_index("c")
    s = lax.axis_index("s")
    iota = lax.iota(jnp.int32, 16)
    rb = rows2.at[0]

    def ldidx(chunk, pt, g, slot):
        pltpu.sync_copy(bsrc.at[pt, chunk, pl.ds(g * G, G)], gsrc2.at[slot])
        pltpu.sync_copy(bdst.at[pt, chunk, pl.ds(g * G, G)], gdst2.at[slot])

    def gather_start(slot):
        @pl.when(slot == 0)
        def _():
            pltpu.async_copy(hs.at[gsrc2.at[0]], rows2.at[0], gsem0)

        @pl.when(slot == 1)
        def _():
            pltpu.async_copy(hs.at[gsrc2.at[1]], rows2.at[1], gsem1)

    def gather_wait(slot):
        @pl.when(slot == 0)
        def _():
            pltpu.make_async_copy(hs.at[gsrc2.at[0]], rows2.at[0], gsem0).wait()

        @pl.when(slot == 1)
        def _():
            pltpu.make_async_copy(hs.at[gsrc2.at[1]], rows2.at[1], gsem1).wait()

    def scatter_start(slot):
        @pl.when(slot == 0)
        def _():
            pltpu.async_copy(rows2.at[0], acc.at[gdst2.at[0]], ssem0, add=True)

        @pl.when(slot == 1)
        def _():
            pltpu.async_copy(rows2.at[1], acc.at[gdst2.at[1]], ssem1, add=True)

    def scatter_wait(slot):
        @pl.when(slot == 0)
        def _():
            pltpu.make_async_copy(rows2.at[0], acc.at[gdst2.at[0]], ssem0).wait()

        @pl.when(slot == 1)
        def _():
            pltpu.make_async_copy(rows2.at[1], acc.at[gdst2.at[1]], ssem1).wait()

    def chunk_body(kk, _):
        chunk = c * (NCHUNK // 2) + kk
        lo = chunk * CH

        ibase = s * TPR
        pltpu.sync_copy(hs.at[pl.ds(lo + ibase, 512)], rb)
        pltpu.sync_copy(rb, acc.at[pl.ds(ibase, 512)])

        @pl.when(s < 15)
        def _init_tail_full():
            pltpu.sync_copy(hs.at[pl.ds(lo + ibase + 512, 272)], rb.at[pl.ds(0, 272)])
            pltpu.sync_copy(rb.at[pl.ds(0, 272)], acc.at[pl.ds(ibase + 512, 272)])

        @pl.when(s == 15)
        def _init_tail_last():
            pltpu.sync_copy(hs.at[pl.ds(lo + ibase + 512, 228)], rb.at[pl.ds(0, 228)])
            pltpu.sync_copy(rb.at[pl.ds(0, 228)], acc.at[pl.ds(ibase + 512, 228)])

        plsc.subcore_barrier()

        lane = c * (NCHUNK // 2) + kk
        for pi in range(2):
            pt = s * 2 + pi
            pltpu.sync_copy(counts.at[pt], cv)
            cvv = cv[...]
            npad = jnp.sum(jnp.where(iota == lane, cvv, 0))
            ng = npad // G

            @pl.when(ng > 0)
            def _prologue():
                ldidx(chunk, pt, 0, 0)
                gather_start(0)

            def grp(g, _):
                slot = g % 2
                nxt = 1 - slot

                @pl.when(g + 1 < ng)
                def _next():
                    @pl.when(g > 0)
                    def _sw():
                        scatter_wait(nxt)

                    ldidx(chunk, pt, g + 1, nxt)
                    gather_start(nxt)

                gather_wait(slot)
                scatter_start(slot)
                return 0

            lax.fori_loop(0, ng, grp, 0)

            @pl.when(ng > 1)
            def _drain2():
                scatter_wait(ng % 2)

            @pl.when(ng > 0)
            def _drain1():
                scatter_wait((ng - 1) % 2)

        plsc.subcore_barrier()

        wbase = s * TPR
        pltpu.sync_copy(acc.at[pl.ds(wbase, 512)], rb)
        pltpu.sync_copy(rb, out.at[pl.ds(lo + wbase, 512)])

        @pl.when(s < 15)
        def _wb_tail_full():
            pltpu.sync_copy(acc.at[pl.ds(wbase + 512, 272)], rb.at[pl.ds(0, 272)])
            pltpu.sync_copy(rb.at[pl.ds(0, 272)], out.at[pl.ds(lo + wbase + 512, 272)])

        @pl.when(s == 15)
        def _wb_tail_last():
            pltpu.sync_copy(acc.at[pl.ds(wbase + 512, 228)], rb.at[pl.ds(0, 228)])
            pltpu.sync_copy(rb.at[pl.ds(0, 228)], out.at[pl.ds(lo + wbase + 512, 228)])

        plsc.subcore_barrier()
        return 0

    lax.fori_loop(0, NCHUNK // 2, chunk_body, 0)


_layer_kernel = functools.partial(
    pl.kernel,
    out_type=jax.ShapeDtypeStruct((N, H), jnp.float32),
    mesh=_mesh,
    scratch_types=[
        pltpu.VMEM((2, G), jnp.int32),
        pltpu.VMEM((2, G), jnp.int32),
        pltpu.VMEM((2, G, H), jnp.float32),
        pltpu.VMEM((16,), jnp.int32),
        pltpu.VMEM_SHARED((CHP, H), jnp.float32),
        pltpu.SemaphoreType.DMA,
        pltpu.SemaphoreType.DMA,
        pltpu.SemaphoreType.DMA,
        pltpu.SemaphoreType.DMA,
    ],
    compiler_params=_sc_params,
)(_layer_body)


BLK = 2000
GRID = N // BLK


def _k1_body(x_ref, W_ref, d0_ref, d1_ref, hs_ref, dinv_ref):
    deg = d0_ref[...] + d1_ref[...] + 1.0
    dinv = lax.rsqrt(deg)
    dinv_ref[...] = dinv
    hs_ref[...] = jnp.dot(x_ref[...], W_ref[...],
                          preferred_element_type=jnp.float32) * dinv


def _k1(x, W1, degp0, degp1):
    return pl.pallas_call(
        _k1_body,
        grid=(GRID,),
        in_specs=[
            pl.BlockSpec((BLK, x.shape[1]), lambda i: (i, 0)),
            pl.BlockSpec(W1.shape, lambda i: (0, 0)),
            pl.BlockSpec((BLK, 1), lambda i: (i, 0)),
            pl.BlockSpec((BLK, 1), lambda i: (i, 0)),
        ],
        out_specs=[
            pl.BlockSpec((BLK, H), lambda i: (i, 0)),
            pl.BlockSpec((BLK, 1), lambda i: (i, 0)),
        ],
        out_shape=[
            jax.ShapeDtypeStruct((N, H), jnp.float32),
            jax.ShapeDtypeStruct((N, 1), jnp.float32),
        ],
    )(x, W1, degp0, degp1)


def _k2_body(part_ref, dinv_ref, b_ref, stats_ref):
    i = pl.program_id(0)
    z = part_ref[...] * dinv_ref[...] + b_ref[...]
    zg = z.reshape(BLK // 8, 8, H)
    s1 = jnp.sum(zg, axis=0)
    s2 = jnp.sum(zg * zg, axis=0)
    st = jnp.stack([s1, s2])

    @pl.when(i == 0)
    def _():
        stats_ref[...] = jnp.zeros_like(stats_ref)

    stats_ref[...] += st


def _k2(part, dinv, b):
    return pl.pallas_call(
        _k2_body,
        grid=(GRID,),
        in_specs=[
            pl.BlockSpec((BLK, H), lambda i: (i, 0)),
            pl.BlockSpec((BLK, 1), lambda i: (i, 0)),
            pl.BlockSpec((1, H), lambda i: (0, 0)),
        ],
        out_specs=pl.BlockSpec((2, 8, H), lambda i: (0, 0, 0)),
        out_shape=jax.ShapeDtypeStruct((2, 8, H), jnp.float32),
    )(part, dinv, b.reshape(1, H))


def _bn_from_stats(z, stats, g, be):
    m = jnp.sum(stats[0], axis=0, keepdims=True) / N
    e2 = jnp.sum(stats[1], axis=0, keepdims=True) / N
    var = e2 - m * m
    return jnp.maximum((z - m) * lax.rsqrt(var + 1e-5) * g + be, 0.0)


def _k3_body(part_ref, dinv_ref, b_ref, stats_ref, g_ref, be_ref, W_ref, hs_ref):
    dinv = dinv_ref[...]
    z = part_ref[...] * dinv + b_ref[...]
    y = _bn_from_stats(z, stats_ref[...], g_ref[...], be_ref[...])
    hs_ref[...] = jnp.dot(y, W_ref[...], preferred_element_type=jnp.float32) * dinv


def _k3(part, dinv, b, stats, g, be, W):
    return pl.pallas_call(
        _k3_body,
        grid=(GRID,),
        in_specs=[
            pl.BlockSpec((BLK, H), lambda i: (i, 0)),
            pl.BlockSpec((BLK, 1), lambda i: (i, 0)),
            pl.BlockSpec((1, H), lambda i: (0, 0)),
            pl.BlockSpec((2, 8, H), lambda i: (0, 0, 0)),
            pl.BlockSpec((1, H), lambda i: (0, 0)),
            pl.BlockSpec((1, H), lambda i: (0, 0)),
            pl.BlockSpec(W.shape, lambda i: (0, 0)),
        ],
        out_specs=pl.BlockSpec((BLK, H), lambda i: (i, 0)),
        out_shape=jax.ShapeDtypeStruct((N, H), jnp.float32),
    )(part, dinv, b.reshape(1, H), stats, g.reshape(1, H), be.reshape(1, H), W)


def _k7_body(part_ref, dinv_ref, b_ref, stats_ref, g_ref, be_ref, batch_ref,
             tW1_ref, tb1_ref, tW2_ref, tb2_ref, lW1_ref, lb1_ref, lW2_ref,
             lb2_ref, sW1_ref, sb1_ref, sW2_ref, sb2_ref,
             t_out, l_out, s_out, pool_acc, cnt_acc):
    i = pl.program_id(0)
    nsteps = pl.num_programs(0)
    z = part_ref[...] * dinv_ref[...] + b_ref[...]
    y = _bn_from_stats(z, stats_ref[...], g_ref[...], be_ref[...])
    bb = batch_ref[...]
    seg = lax.broadcasted_iota(jnp.int32, (BLK, B), 1)
    onehot = (bb == seg).astype(jnp.float32)
    psum = lax.dot_general(onehot, y, (((0,), (0,)), ((), ())))
    ones = jnp.ones((BLK, 8), jnp.float32)
    csum = lax.dot_general(onehot, ones, (((0,), (0,)), ((), ())))

    @pl.when(i == 0)
    def _init():
        pool_acc[...] = jnp.zeros_like(pool_acc)
        cnt_acc[...] = jnp.zeros_like(cnt_acc)

    pool_acc[...] += psum
    cnt_acc[...] += csum

    @pl.when(i == nsteps - 1)
    def _final():
        counts = jnp.clip(cnt_acc[...][:, 0:1], 1.0, None)
        emb = pool_acc[...] / counts
        th = jnp.maximum(jnp.dot(emb, tW1_ref[...], preferred_element_type=jnp.float32) + tb1_ref[...], 0.0)
        t_out[...] = jnp.dot(th, tW2_ref[...], preferred_element_type=jnp.float32) + tb2_ref[...]
        lh = jnp.maximum(jnp.dot(emb, lW1_ref[...], preferred_element_type=jnp.float32) + lb1_ref[...], 0.0)
        l_out[...] = jax.nn.sigmoid(jnp.dot(lh, lW2_ref[...], preferred_element_type=jnp.float32) + lb2_ref[...])
        sh = jnp.maximum(jnp.dot(emb, sW1_ref[...], preferred_element_type=jnp.float32) + sb1_ref[...], 0.0)
        s_out[...] = jax.nn.sigmoid(jnp.dot(sh, sW2_ref[...], preferred_element_type=jnp.float32) + sb2_ref[...])


def _k7(part, dinv, b, stats, g, be, batch,
        tW1, tb1, tW2, tb2, lW1, lb1, lW2, lb2, sW1, sb1, sW2, sb2):
    full = lambda shape: pl.BlockSpec(shape, lambda i: (0,) * len(shape))
    return pl.pallas_call(
        _k7_body,
        grid=(GRID,),
        in_specs=[
            pl.BlockSpec((BLK, H), lambda i: (i, 0)),
            pl.BlockSpec((BLK, 1), lambda i: (i, 0)),
            full((1, H)),
            full((2, 8, H)),
            full((1, H)),
            full((1, H)),
            pl.BlockSpec((BLK, 1), lambda i: (i, 0)),
            full(tW1.shape), full((1, tb1.shape[0])), full(tW2.shape), full((1, tb2.shape[0])),
            full(lW1.shape), full((1, lb1.shape[0])), full(lW2.shape), full((1, lb2.shape[0])),
            full(sW1.shape), full((1, sb1.shape[0])), full(sW2.shape), full((1, sb2.shape[0])),
        ],
        out_specs=[full((B, 6)), full((B, 2)), full((B, 1))],
        out_shape=[
            jax.ShapeDtypeStruct((B, 6), jnp.float32),
            jax.ShapeDtypeStruct((B, 2), jnp.float32),
            jax.ShapeDtypeStruct((B, 1), jnp.float32),
        ],
        scratch_shapes=[
            pltpu.VMEM((B, H), jnp.float32),
            pltpu.VMEM((B, 8), jnp.float32),
        ],
    )(part, dinv, b.reshape(1, H), stats, g.reshape(1, H), be.reshape(1, H),
      batch.reshape(N, 1), tW1, tb1.reshape(1, -1), tW2, tb2.reshape(1, -1),
      lW1, lb1.reshape(1, -1), lW2, lb2.reshape(1, -1),
      sW1, sb1.reshape(1, -1), sW2, sb2.reshape(1, -1))


def kernel(x, edge_index, batch, W1, b1, W2, b2, W3, b3, g1, be1, g2, be2, g3, be3,
           tW1, tb1, tW2, tb2, lW1, lb1, lW2, lb2, sW1, sb1, sW2, sb2):
    srcs = edge_index[0]
    dsts = edge_index[1]

    bsrc, bdst, counts, degp0, degp1 = _bin_kernel(dsts, srcs)
    hs1, dinv = _k1(x, W1, degp0.reshape(N, 1), degp1.reshape(N, 1))

    part1 = _layer_kernel(hs1, bsrc, bdst, counts)
    st1 = _k2(part1, dinv, b1)
    hs2 = _k3(part1, dinv, b1, st1, g1, be1, W2)

    part2 = _layer_kernel(hs2, bsrc, bdst, counts)
    st2 = _k2(part2, dinv, b2)
    hs3 = _k3(part2, dinv, b2, st2, g2, be2, W3)

    part3 = _layer_kernel(hs3, bsrc, bdst, counts)
    st3 = _k2(part3, dinv, b3)
    return _k7(part3, dinv, b3, st3, g3, be3, batch,
               tW1, tb1, tW2, tb2, lW1, lb1, lW2, lb2, sW1, sb1, sW2, sb2)


# speedup vs baseline: 1.0479x; 1.0479x over previous
import functools

import jax
import jax.numpy as jnp
from jax import lax
from jax.experimental import pallas as pl
from jax.experimental.pallas import tpu as pltpu
from jax.experimental.pallas import tpu_sc as plsc

N = 100000
E = 1600000
H = 64
B = 64

NCHUNK = 8
CH = N // NCHUNK
TPR = 784
CHP = 16 * TPR
G = 512
SB = 2000
ET32 = E // 32
STAG = 2560
BCAP = ((ET32 + G - 1) // G) * G

_mesh = plsc.VectorSubcoreMesh(core_axis_name="c", subcore_axis_name="s")
_sc_params = pltpu.CompilerParams(needs_layout_passes=False,
                                  use_tc_tiling_on_sc=False)


def _bin_body(dst_hbm, src_hbm,
              bsrc, bdst, counts, deg_out0, deg_out1,
              dstbuf, srcbuf,
              st_s0, st_s1, st_s2, st_s3, st_s4, st_s5, st_s6, st_s7,
              st_d0, st_d1, st_d2, st_d3, st_d4, st_d5, st_d6, st_d7,
              ones_v, zbuf, cntv, deg_sp, sem):
    c = lax.axis_index("c")
    s = lax.axis_index("s")
    wid = s * 2 + c
    iota = lax.iota(jnp.int32, 16)
    st_s = [st_s0, st_s1, st_s2, st_s3, st_s4, st_s5, st_s6, st_s7]
    st_d = [st_d0, st_d1, st_d2, st_d3, st_d4, st_d5, st_d6, st_d7]

    def _fill(j, _):
        zbuf[pl.ds(j * 16, 16)] = jnp.zeros((16,), jnp.float32)
        ones_v[pl.ds(j * 16, 16)] = jnp.ones((16,), jnp.float32)
        return 0

    lax.fori_loop(0, SB // 16, _fill, 0)

    NBLK = N // SB
    for k in range((NBLK + 15) // 16):
        blk = 16 * k

        @pl.when(blk + s < NBLK)
        def _z():
            pltpu.sync_copy(zbuf, deg_sp.at[pl.ds((blk + s) * SB, SB)])

    plsc.subcore_barrier()

    ebase = wid * ET32

    def vloop(v, carry):
        d = dstbuf[pl.ds(v * 16, 16)]
        sv = srcbuf[pl.ds(v * 16, 16)]
        out = []
        for k in range(NCHUNK):
            lo_v = jnp.full((16,), k * CH, jnp.int32)
            hi_v = jnp.full((16,), (k + 1) * CH, jnp.int32)
            m = (d >= lo_v) & (d < hi_v)
            mi = jnp.where(m, 1, 0)
            cs = plsc.cumsum(mi)
            pos = jnp.full((16,), carry[k], jnp.int32) + cs - 1
            plsc.store_scatter(st_d[k], [pos], d - lo_v, mask=m)
            plsc.store_scatter(st_s[k], [pos], sv, mask=m)
            out.append(carry[k] + cs[15])
        return tuple(out)

    def block(b, carry):
        pltpu.sync_copy(dst_hbm.at[pl.ds(ebase + b * SB, SB)], dstbuf)
        pltpu.sync_copy(src_hbm.at[pl.ds(ebase + b * SB, SB)], srcbuf)
        pltpu.sync_copy(ones_v, deg_sp.at[dstbuf], add=True)
        leads = lax.fori_loop(0, SB // 16, vloop, carry[:NCHUNK])
        new = []
        ngs = []
        for k in range(NCHUNK):
            cnt = leads[k]
            ng = cnt // G

            def flush(g, _, k=k):
                off = (carry[NCHUNK + k] + g) * G
                pltpu.sync_copy(st_s[k].at[pl.ds(g * G, G)],
                                bsrc.at[wid, k, pl.ds(off, G)])
                pltpu.sync_copy(st_d[k].at[pl.ds(g * G, G)],
                                bdst.at[wid, k, pl.ds(off, G)])
                return 0

            lax.fori_loop(0, ng, flush, 0)

            def mv(j, _, k=k):
                st_s[k][pl.ds(j * 16, 16)] = st_s[k][pl.ds(ng * G + j * 16, 16)]
                st_d[k][pl.ds(j * 16, 16)] = st_d[k][pl.ds(ng * G + j * 16, 16)]
                return 0

            lax.fori_loop(0, G // 16, mv, 0)
            new.append(cnt - ng * G)
            ngs.append(carry[NCHUNK + k] + ng)
        return tuple(new) + tuple(ngs)

    carry = lax.fori_loop(0, ET32 // SB, block,
                          (jnp.int32(0),) * (2 * NCHUNK))

    pad_src = iota + jnp.full((16,), wid * 64, jnp.int32)
    totals = []
    for k in range(NCHUNK):
        lead = carry[k]
        ngroups = carry[NCHUNK + k]
        pad_dst = (iota & 7) + jnp.full((16,), CH + 8 * (wid & 1), jnp.int32)

        @pl.when(lead > 0)
        def _final(k=k, lead=lead, ngroups=ngroups, pad_dst=pad_dst):
            for j in range(G // 16):
                st_d[k][pl.ds(lead + j * 16, 16)] = pad_dst
                st_s[k][pl.ds(lead + j * 16, 16)] = pad_src
            off = ngroups * G
            pltpu.sync_copy(st_s[k].at[pl.ds(0, G)], bsrc.at[wid, k, pl.ds(off, G)])
            pltpu.sync_copy(st_d[k].at[pl.ds(0, G)], bdst.at[wid, k, pl.ds(off, G)])

        totals.append((ngroups + jnp.where(lead > 0, 1, 0)) * G)

    cv = jnp.full((16,), 0, jnp.int32)
    for k in range(NCHUNK):
        cv = jnp.where(iota == k, jnp.full((16,), totals[k], jnp.int32), cv)
    cntv[...] = cv
    pltpu.sync_copy(cntv, counts.at[wid])

    plsc.subcore_barrier()
    for k in range((NBLK + 15) // 16):
        blk = 16 * k

        @pl.when(blk + s < NBLK)
        def _stage():
            pltpu.sync_copy(deg_sp.at[pl.ds((blk + s) * SB, SB)], zbuf)

        @pl.when((blk + s < NBLK) & (c == 0))
        def _wb0():
            pltpu.sync_copy(zbuf, deg_out0.at[pl.ds((blk + s) * SB, SB)])

        @pl.when((blk + s < NBLK) & (c == 1))
        def _wb1():
            pltpu.sync_copy(zbuf, deg_out1.at[pl.ds((blk + s) * SB, SB)])


_bin_kernel = functools.partial(
    pl.kernel,
    out_type=[
        jax.ShapeDtypeStruct((32, NCHUNK, BCAP), jnp.int32),
        jax.ShapeDtypeStruct((32, NCHUNK, BCAP), jnp.int32),
        jax.ShapeDtypeStruct((32, 16), jnp.int32),
        jax.ShapeDtypeStruct((N,), jnp.float32),
        jax.ShapeDtypeStruct((N,), jnp.float32),
    ],
    mesh=_mesh,
    scratch_types=[
        pltpu.VMEM((SB,), jnp.int32),
        pltpu.VMEM((SB,), jnp.int32),
    ] + [pltpu.VMEM((STAG,), jnp.int32)] * 16 + [
        pltpu.VMEM((SB,), jnp.float32),
        pltpu.VMEM((SB,), jnp.float32),
        pltpu.VMEM((16,), jnp.int32),
        pltpu.VMEM_SHARED((N,), jnp.float32),
        pltpu.SemaphoreType.DMA,
    ],
    compiler_params=_sc_params,
)(_bin_body)


def _layer_body(hs, bsrc, bdst, counts, out,
                gsrc2, gdst2, rows2, cv, acc, gsem0, gsem1, ssem0, ssem1):
    c = lax.axis_index("c")
    s = lax.axis_index("s")
    iota = lax.iota(jnp.int32, 16)
    rb = rows2.at[0]

    def ldidx(chunk, pt, g, slot):
        pltpu.sync_copy(bsrc.at[pt, chunk, pl.ds(g * G, G)], gsrc2.at[slot])
        pltpu.sync_copy(bdst.at[pt, chunk, pl.ds(g * G, G)], gdst2.at[slot])

    def gather_start(slot):
        @pl.when(slot == 0)
        def _():
            pltpu.async_copy(hs.at[gsrc2.at[0]], rows2.at[0], gsem0)

        @pl.when(slot == 1)
        def _():
            pltpu.async_copy(hs.at[gsrc2.at[1]], rows2.at[1], gsem1)

    def gather_wait(slot):
        @pl.when(slot == 0)
        def _():
            pltpu.make_async_copy(hs.at[gsrc2.at[0]], rows2.at[0], gsem0).wait()

        @pl.when(slot == 1)
        def _():
            pltpu.make_async_copy(hs.at[gsrc2.at[1]], rows2.at[1], gsem1).wait()

    def scatter_start(slot):
        @pl.when(slot == 0)
        def _():
            pltpu.async_copy(rows2.at[0], acc.at[gdst2.at[0]], ssem0, add=True)

        @pl.when(slot == 1)
        def _():
            pltpu.async_copy(rows2.at[1], acc.at[gdst2.at[1]], ssem1, add=True)

    def scatter_wait(slot):
        @pl.when(slot == 0)
        def _():
            pltpu.make_async_copy(rows2.at[0], acc.at[gdst2.at[0]], ssem0).wait()

        @pl.when(slot == 1)
        def _():
            pltpu.make_async_copy(rows2.at[1], acc.at[gdst2.at[1]], ssem1).wait()

    def chunk_body(kk, _):
        chunk = c * (NCHUNK // 2) + kk
        lo = chunk * CH

        ibase = s * TPR
        pltpu.sync_copy(hs.at[pl.ds(lo + ibase, 512)], rb)
        pltpu.sync_copy(rb, acc.at[pl.ds(ibase, 512)])

        @pl.when(s < 15)
        def _init_tail_full():
            pltpu.sync_copy(hs.at[pl.ds(lo + ibase + 512, 272)], rb.at[pl.ds(0, 272)])
            pltpu.sync_copy(rb.at[pl.ds(0, 272)], acc.at[pl.ds(ibase + 512, 272)])

        @pl.when(s == 15)
        def _init_tail_last():
            pltpu.sync_copy(hs.at[pl.ds(lo + ibase + 512, 228)], rb.at[pl.ds(0, 228)])
            pltpu.sync_copy(rb.at[pl.ds(0, 228)], acc.at[pl.ds(ibase + 512, 228)])

        plsc.subcore_barrier()

        lane = c * (NCHUNK // 2) + kk
        for pi in range(2):
            pt = s * 2 + pi
            pltpu.sync_copy(counts.at[pt], cv)
            cvv = cv[...]
            npad = jnp.sum(jnp.where(iota == lane, cvv, 0))
            ng = npad // G

            @pl.when(ng > 0)
            def _prologue():
                ldidx(chunk, pt, 0, 0)
                gather_start(0)

            def grp(g, _):
                slot = g % 2
                nxt = 1 - slot

                @pl.when(g + 1 < ng)
                def _next():
                    @pl.when(g > 0)
                    def _sw():
                        scatter_wait(nxt)

                    ldidx(chunk, pt, g + 1, nxt)
                    gather_start(nxt)

                gather_wait(slot)
                scatter_start(slot)
                return 0

            lax.fori_loop(0, ng, grp, 0)

            @pl.when(ng > 1)
            def _drain2():
                scatter_wait(ng % 2)

            @pl.when(ng > 0)
            def _drain1():
                scatter_wait((ng - 1) % 2)

        plsc.subcore_barrier()

        wbase = s * TPR
        pltpu.sync_copy(acc.at[pl.ds(wbase, 512)], rb)
        pltpu.sync_copy(rb, out.at[pl.ds(lo + wbase, 512)])

        @pl.when(s < 15)
        def _wb_tail_full():
            pltpu.sync_copy(acc.at[pl.ds(wbase + 512, 272)], rb.at[pl.ds(0, 272)])
            pltpu.sync_copy(rb.at[pl.ds(0, 272)], out.at[pl.ds(lo + wbase + 512, 272)])

        @pl.when(s == 15)
        def _wb_tail_last():
            pltpu.sync_copy(acc.at[pl.ds(wbase + 512, 228)], rb.at[pl.ds(0, 228)])
            pltpu.sync_copy(rb.at[pl.ds(0, 228)], out.at[pl.ds(lo + wbase + 512, 228)])

        plsc.subcore_barrier()
        return 0

    lax.fori_loop(0, NCHUNK // 2, chunk_body, 0)


_layer_kernel = functools.partial(
    pl.kernel,
    out_type=jax.ShapeDtypeStruct((N, H), jnp.float32),
    mesh=_mesh,
    scratch_types=[
        pltpu.VMEM((2, G), jnp.int32),
        pltpu.VMEM((2, G), jnp.int32),
        pltpu.VMEM((2, G, H), jnp.float32),
        pltpu.VMEM((16,), jnp.int32),
        pltpu.VMEM_SHARED((CHP, H), jnp.float32),
        pltpu.SemaphoreType.DMA,
        pltpu.SemaphoreType.DMA,
        pltpu.SemaphoreType.DMA,
        pltpu.SemaphoreType.DMA,
    ],
    compiler_params=_sc_params,
)(_layer_body)


BLK = 5000
GRID = N // BLK


def _k1_body(x_ref, W_ref, d0_ref, d1_ref, hs_ref, dinv_ref):
    deg = d0_ref[...] + d1_ref[...] + 1.0
    dinv = lax.rsqrt(deg)
    dinv_ref[...] = dinv
    hs_ref[...] = jnp.dot(x_ref[...], W_ref[...],
                          preferred_element_type=jnp.float32) * dinv


def _k1(x, W1, degp0, degp1):
    return pl.pallas_call(
        _k1_body,
        grid=(GRID,),
        in_specs=[
            pl.BlockSpec((BLK, x.shape[1]), lambda i: (i, 0)),
            pl.BlockSpec(W1.shape, lambda i: (0, 0)),
            pl.BlockSpec((BLK, 1), lambda i: (i, 0)),
            pl.BlockSpec((BLK, 1), lambda i: (i, 0)),
        ],
        out_specs=[
            pl.BlockSpec((BLK, H), lambda i: (i, 0)),
            pl.BlockSpec((BLK, 1), lambda i: (i, 0)),
        ],
        out_shape=[
            jax.ShapeDtypeStruct((N, H), jnp.float32),
            jax.ShapeDtypeStruct((N, 1), jnp.float32),
        ],
    )(x, W1, degp0, degp1)


def _k2_body(part_ref, dinv_ref, b_ref, stats_ref):
    i = pl.program_id(0)
    z = part_ref[...] * dinv_ref[...] + b_ref[...]
    zg = z.reshape(BLK // 8, 8, H)
    s1 = jnp.sum(zg, axis=0)
    s2 = jnp.sum(zg * zg, axis=0)
    st = jnp.stack([s1, s2])

    @pl.when(i == 0)
    def _():
        stats_ref[...] = jnp.zeros_like(stats_ref)

    stats_ref[...] += st


def _k2(part, dinv, b):
    return pl.pallas_call(
        _k2_body,
        grid=(GRID,),
        in_specs=[
            pl.BlockSpec((BLK, H), lambda i: (i, 0)),
            pl.BlockSpec((BLK, 1), lambda i: (i, 0)),
            pl.BlockSpec((1, H), lambda i: (0, 0)),
        ],
        out_specs=pl.BlockSpec((2, 8, H), lambda i: (0, 0, 0)),
        out_shape=jax.ShapeDtypeStruct((2, 8, H), jnp.float32),
    )(part, dinv, b.reshape(1, H))


def _bn_from_stats(z, stats, g, be):
    m = jnp.sum(stats[0], axis=0, keepdims=True) / N
    e2 = jnp.sum(stats[1], axis=0, keepdims=True) / N
    var = e2 - m * m
    return jnp.maximum((z - m) * lax.rsqrt(var + 1e-5) * g + be, 0.0)


def _k3_body(part_ref, dinv_ref, b_ref, stats_ref, g_ref, be_ref, W_ref, hs_ref):
    dinv = dinv_ref[...]
    z = part_ref[...] * dinv + b_ref[...]
    y = _bn_from_stats(z, stats_ref[...], g_ref[...], be_ref[...])
    hs_ref[...] = jnp.dot(y, W_ref[...], preferred_element_type=jnp.float32) * dinv


def _k3(part, dinv, b, stats, g, be, W):
    return pl.pallas_call(
        _k3_body,
        grid=(GRID,),
        in_specs=[
            pl.BlockSpec((BLK, H), lambda i: (i, 0)),
            pl.BlockSpec((BLK, 1), lambda i: (i, 0)),
            pl.BlockSpec((1, H), lambda i: (0, 0)),
            pl.BlockSpec((2, 8, H), lambda i: (0, 0, 0)),
            pl.BlockSpec((1, H), lambda i: (0, 0)),
            pl.BlockSpec((1, H), lambda i: (0, 0)),
            pl.BlockSpec(W.shape, lambda i: (0, 0)),
        ],
        out_specs=pl.BlockSpec((BLK, H), lambda i: (i, 0)),
        out_shape=jax.ShapeDtypeStruct((N, H), jnp.float32),
    )(part, dinv, b.reshape(1, H), stats, g.reshape(1, H), be.reshape(1, H), W)


def _k7_body(part_ref, dinv_ref, b_ref, stats_ref, g_ref, be_ref, batch_ref,
             tW1_ref, tb1_ref, tW2_ref, tb2_ref, lW1_ref, lb1_ref, lW2_ref,
             lb2_ref, sW1_ref, sb1_ref, sW2_ref, sb2_ref,
             t_out, l_out, s_out, pool_acc, cnt_acc):
    i = pl.program_id(0)
    nsteps = pl.num_programs(0)
    z = part_ref[...] * dinv_ref[...] + b_ref[...]
    y = _bn_from_stats(z, stats_ref[...], g_ref[...], be_ref[...])
    bb = batch_ref[...]
    seg = lax.broadcasted_iota(jnp.int32, (BLK, B), 1)
    onehot = (bb == seg).astype(jnp.float32)
    psum = lax.dot_general(onehot, y, (((0,), (0,)), ((), ())))
    ones = jnp.ones((BLK, 8), jnp.float32)
    csum = lax.dot_general(onehot, ones, (((0,), (0,)), ((), ())))

    @pl.when(i == 0)
    def _init():
        pool_acc[...] = jnp.zeros_like(pool_acc)
        cnt_acc[...] = jnp.zeros_like(cnt_acc)

    pool_acc[...] += psum
    cnt_acc[...] += csum

    @pl.when(i == nsteps - 1)
    def _final():
        counts = jnp.clip(cnt_acc[...][:, 0:1], 1.0, None)
        emb = pool_acc[...] / counts
        th = jnp.maximum(jnp.dot(emb, tW1_ref[...], preferred_element_type=jnp.float32) + tb1_ref[...], 0.0)
        t_out[...] = jnp.dot(th, tW2_ref[...], preferred_element_type=jnp.float32) + tb2_ref[...]
        lh = jnp.maximum(jnp.dot(emb, lW1_ref[...], preferred_element_type=jnp.float32) + lb1_ref[...], 0.0)
        l_out[...] = jax.nn.sigmoid(jnp.dot(lh, lW2_ref[...], preferred_element_type=jnp.float32) + lb2_ref[...])
        sh = jnp.maximum(jnp.dot(emb, sW1_ref[...], preferred_element_type=jnp.float32) + sb1_ref[...], 0.0)
        s_out[...] = jax.nn.sigmoid(jnp.dot(sh, sW2_ref[...], preferred_element_type=jnp.float32) + sb2_ref[...])


def _k7(part, dinv, b, stats, g, be, batch,
        tW1, tb1, tW2, tb2, lW1, lb1, lW2, lb2, sW1, sb1, sW2, sb2):
    full = lambda shape: pl.BlockSpec(shape, lambda i: (0,) * len(shape))
    return pl.pallas_call(
        _k7_body,
        grid=(GRID,),
        in_specs=[
            pl.BlockSpec((BLK, H), lambda i: (i, 0)),
            pl.BlockSpec((BLK, 1), lambda i: (i, 0)),
            full((1, H)),
            full((2, 8, H)),
            full((1, H)),
            full((1, H)),
            pl.BlockSpec((BLK, 1), lambda i: (i, 0)),
            full(tW1.shape), full((1, tb1.shape[0])), full(tW2.shape), full((1, tb2.shape[0])),
            full(lW1.shape), full((1, lb1.shape[0])), full(lW2.shape), full((1, lb2.shape[0])),
            full(sW1.shape), full((1, sb1.shape[0])), full(sW2.shape), full((1, sb2.shape[0])),
        ],
        out_specs=[full((B, 6)), full((B, 2)), full((B, 1))],
        out_shape=[
            jax.ShapeDtypeStruct((B, 6), jnp.float32),
            jax.ShapeDtypeStruct((B, 2), jnp.float32),
            jax.ShapeDtypeStruct((B, 1), jnp.float32),
        ],
        scratch_shapes=[
            pltpu.VMEM((B, H), jnp.float32),
            pltpu.VMEM((B, 8), jnp.float32),
        ],
    )(part, dinv, b.reshape(1, H), stats, g.reshape(1, H), be.reshape(1, H),
      batch.reshape(N, 1), tW1, tb1.reshape(1, -1), tW2, tb2.reshape(1, -1),
      lW1, lb1.reshape(1, -1), lW2, lb2.reshape(1, -1),
      sW1, sb1.reshape(1, -1), sW2, sb2.reshape(1, -1))


def kernel(x, edge_index, batch, W1, b1, W2, b2, W3, b3, g1, be1, g2, be2, g3, be3,
           tW1, tb1, tW2, tb2, lW1, lb1, lW2, lb2, sW1, sb1, sW2, sb2):
    srcs = edge_index[0]
    dsts = edge_index[1]

    bsrc, bdst, counts, degp0, degp1 = _bin_kernel(dsts, srcs)
    hs1, dinv = _k1(x, W1, degp0.reshape(N, 1), degp1.reshape(N, 1))

    part1 = _layer_kernel(hs1, bsrc, bdst, counts)
    st1 = _k2(part1, dinv, b1)
    hs2 = _k3(part1, dinv, b1, st1, g1, be1, W2)

    part2 = _layer_kernel(hs2, bsrc, bdst, counts)
    st2 = _k2(part2, dinv, b2)
    hs3 = _k3(part2, dinv, b2, st2, g2, be2, W3)

    part3 = _layer_kernel(hs3, bsrc, bdst, counts)
    st3 = _k2(part3, dinv, b3)
    return _k7(part3, dinv, b3, st3, g3, be3, batch,
               tW1, tb1, tW2, tb2, lW1, lb1, lW2, lb2, sW1, sb1, sW2, sb2)
